# baseline (speedup 1.0000x reference)
import functools

import jax
import jax.numpy as jnp
from jax import lax
from jax.experimental import pallas as pl
from jax.experimental.pallas import tpu as pltpu
from jax.experimental.pallas import tpu_sc as plsc

D = 1024
N_ROWS = 16384
N_SC = 8192
NW = 32
ROWS_PER_W = N_SC // NW
C = 16
N_CHUNKS = ROWS_PER_W // C
LANES = 16
CBLKS = D // LANES
NBUF = 4
PD = 2

TC_ROWS_PER_BLOCK = 512
TC_BLOCKS = (N_ROWS - N_SC) // TC_ROWS_PER_BLOCK


def _sc_body(x_hbm, fs_hbm, tab_hbm, out_hbm,
             xbuf, sbuf, tbuf, semx, semi, semo):
    wid = lax.axis_index("s") * 2 + lax.axis_index("c")
    base = wid * ROWS_PER_W

    pltpu.sync_copy(tab_hbm, tbuf)

    def in_descs(g, b):
        row0 = base + g * C
        return (
            pltpu.make_async_copy(x_hbm.at[pl.ds(row0, C)], xbuf.at[b],
                                  semx.at[b]),
            pltpu.make_async_copy(fs_hbm.at[pl.ds(row0, C)], sbuf.at[b],
                                  semi.at[b]),
        )

    def out_desc(g, b):
        row0 = base + g * C
        return pltpu.make_async_copy(xbuf.at[b], out_hbm.at[pl.ds(row0, C)],
                                     semo.at[b])

    def start_in(g, b):
        for d in in_descs(g, b):
            d.start()

    def wait_in(g, b):
        for d in in_descs(g, b):
            d.wait()

    def compute(b):
        @plsc.parallel_loop(0, C, 2)
        def row_body(r):
            fa = sbuf[b, r]
            fb = sbuf[b, r + 1]

            @plsc.parallel_loop(0, CBLKS, 1, unroll=8)
            def blk_body(c):
                off = c * LANES
                ta = plsc.load_gather(tbuf, [fa + off])
                tb = plsc.load_gather(tbuf, [fb + off])
                plsc.addupdate(xbuf.at[b, r, pl.ds(off, LANES)], ta)
                plsc.addupdate(xbuf.at[b, r + 1, pl.ds(off, LANES)], tb)

    for g in range(PD):
        start_in(g, g % NBUF)
    for g in range(PD):
        b = g % NBUF
        wait_in(g, b)
        gp = g + PD
        bp = gp % NBUF
        if gp >= NBUF:
            out_desc(gp - NBUF, bp).wait()
        start_in(gp, bp)
        compute(b)
        out_desc(g, b).start()

    def main_body(kk, _):
        for off in range(NBUF):
            g = kk * NBUF + PD + off
            b = (PD + off) % NBUF
            bp = (PD + off + PD) % NBUF
            wait_in(g, b)
            out_desc(g + PD - NBUF, bp).wait()
            start_in(g + PD, bp)
            compute(b)
            out_desc(g, b).start()
        return 0

    lax.fori_loop(0, (N_CHUNKS - 2 * PD) // NBUF, main_body, 0)

    for g in range(N_CHUNKS - PD, N_CHUNKS):
        b = g % NBUF
        wait_in(g, b)
        compute(b)
        out_desc(g, b).start()

    for g in range(N_CHUNKS - NBUF, N_CHUNKS):
        out_desc(g, g % NBUF).wait()


def _tc_body(idx_ref, x_ref, tab_ref, out_ref):
    idx = idx_ref[0, 0]
    idxc = idx[:, None]
    t0 = tab_ref[0][None, :]
    t1 = tab_ref[1][None, :]
    t2 = tab_ref[2][None, :]
    emb = jnp.where(idxc == 0, t0, jnp.where(idxc == 1, t1, t2))
    out_ref[...] = x_ref[...] + emb


def kernel(x, type_idx, type_embedding):
    B, S, d = x.shape
    x2 = x.reshape(N_ROWS, D)
    idx = type_idx.reshape(N_ROWS).astype(jnp.int32)
    fsplat = idx[:, None] * D + jnp.arange(LANES, dtype=jnp.int32)[None, :]
    tab = type_embedding.reshape(3 * D)

    mesh = plsc.VectorSubcoreMesh(core_axis_name="c", subcore_axis_name="s")
    sc_f = functools.partial(
        pl.kernel,
        out_type=jax.ShapeDtypeStruct((N_SC, D), jnp.float32),
        mesh=mesh,
        compiler_params=pltpu.CompilerParams(needs_layout_passes=False),
        scratch_types=[
            pltpu.VMEM((NBUF, C, D), jnp.float32),
            pltpu.VMEM((NBUF, C, LANES), jnp.int32),
            pltpu.VMEM((3 * D,), jnp.float32),
            pltpu.SemaphoreType.DMA((NBUF,)),
            pltpu.SemaphoreType.DMA((NBUF,)),
            pltpu.SemaphoreType.DMA((NBUF,)),
        ],
    )(_sc_body)
    sc_out = sc_f(x2, fsplat, tab)

    nblk_all = N_ROWS // TC_ROWS_PER_BLOCK
    x3 = x2.reshape(nblk_all, TC_ROWS_PER_BLOCK, D)
    idx3 = idx.reshape(nblk_all, 1, TC_ROWS_PER_BLOCK)
    blk0 = N_SC // TC_ROWS_PER_BLOCK
    tc_out = pl.pallas_call(
        _tc_body,
        grid=(TC_BLOCKS,),
        in_specs=[
            pl.BlockSpec((1, 1, TC_ROWS_PER_BLOCK), lambda i: (i + blk0, 0, 0)),
            pl.BlockSpec((1, TC_ROWS_PER_BLOCK, D), lambda i: (i + blk0, 0, 0)),
            pl.BlockSpec((3, D), lambda i: (0, 0)),
        ],
        out_specs=pl.BlockSpec((1, TC_ROWS_PER_BLOCK, D), lambda i: (i, 0, 0)),
        out_shape=jax.ShapeDtypeStruct((TC_BLOCKS, TC_ROWS_PER_BLOCK, D),
                                       x.dtype),
    )(idx3, x3, type_embedding)

    out = jnp.concatenate(
        [sc_out, tc_out.reshape(N_ROWS - N_SC, D)], axis=0)
    return out.reshape(B, S, d)


# speedup vs baseline: 1.4602x; 1.4602x over previous
import functools

import jax
import jax.numpy as jnp
from jax import lax
from jax.experimental import pallas as pl
from jax.experimental.pallas import tpu as pltpu
from jax.experimental.pallas import tpu_sc as plsc

D = 1024
N_ROWS = 16384
NW = 32
ROWS_PER_W = N_ROWS // NW
C = 16
N_CHUNKS = ROWS_PER_W // C
LANES = 16
CBLKS = D // LANES
NBUF = 6
PD = 4


def _sc_body(x_hbm, fs_hbm, tab_hbm, out_hbm,
             xbuf, sbuf, tbuf, semx, semi, semo):
    wid = lax.axis_index("s") * 2 + lax.axis_index("c")
    base = wid * ROWS_PER_W

    pltpu.sync_copy(tab_hbm, tbuf)

    def in_descs(g, b):
        row0 = base + g * C
        return (
            pltpu.make_async_copy(x_hbm.at[pl.ds(row0, C)], xbuf.at[b],
                                  semx.at[b]),
            pltpu.make_async_copy(fs_hbm.at[pl.ds(row0, C)], sbuf.at[b],
                                  semi.at[b]),
        )

    def out_desc(g, b):
        row0 = base + g * C
        return pltpu.make_async_copy(xbuf.at[b], out_hbm.at[pl.ds(row0, C)],
                                     semo.at[b])

    def start_in(g, b):
        for d in in_descs(g, b):
            d.start()

    def wait_in(g, b):
        for d in in_descs(g, b):
            d.wait()

    def compute(b):
        @plsc.parallel_loop(0, C, 2)
        def row_body(r):
            fa = sbuf[b, r]
            fb = sbuf[b, r + 1]

            @plsc.parallel_loop(0, CBLKS, 1, unroll=8)
            def blk_body(c):
                off = c * LANES
                ta = plsc.load_gather(tbuf, [fa + off])
                tb = plsc.load_gather(tbuf, [fb + off])
                plsc.addupdate(xbuf.at[b, r, pl.ds(off, LANES)], ta)
                plsc.addupdate(xbuf.at[b, r + 1, pl.ds(off, LANES)], tb)

    for g in range(PD):
        start_in(g, g % NBUF)
    for g in range(PD):
        b = g % NBUF
        wait_in(g, b)
        gp = g + PD
        bp = gp % NBUF
        if gp >= NBUF:
            out_desc(gp - NBUF, bp).wait()
        start_in(gp, bp)
        compute(b)
        out_desc(g, b).start()

    def main_body(kk, _):
        for off in range(NBUF):
            g = kk * NBUF + PD + off
            b = (PD + off) % NBUF
            bp = (PD + off + PD) % NBUF
            wait_in(g, b)
            out_desc(g + PD - NBUF, bp).wait()
            start_in(g + PD, bp)
            compute(b)
            out_desc(g, b).start()
        return 0

    lax.fori_loop(0, (N_CHUNKS - 2 * PD) // NBUF, main_body, 0)

    for g in range(N_CHUNKS - PD, N_CHUNKS):
        b = g % NBUF
        wait_in(g, b)
        compute(b)
        out_desc(g, b).start()

    for g in range(N_CHUNKS - NBUF, N_CHUNKS):
        out_desc(g, g % NBUF).wait()


def kernel(x, type_idx, type_embedding):
    B, S, d = x.shape
    x2 = x.reshape(N_ROWS, D)
    idx = type_idx.reshape(N_ROWS).astype(jnp.int32)
    fsplat = idx[:, None] * D + jnp.arange(LANES, dtype=jnp.int32)[None, :]
    tab = type_embedding.reshape(3 * D)

    mesh = plsc.VectorSubcoreMesh(core_axis_name="c", subcore_axis_name="s")
    f = functools.partial(
        pl.kernel,
        out_type=jax.ShapeDtypeStruct((N_ROWS, D), jnp.float32),
        mesh=mesh,
        compiler_params=pltpu.CompilerParams(needs_layout_passes=False),
        scratch_types=[
            pltpu.VMEM((NBUF, C, D), jnp.float32),
            pltpu.VMEM((NBUF, C, LANES), jnp.int32),
            pltpu.VMEM((3 * D,), jnp.float32),
            pltpu.SemaphoreType.DMA((NBUF,)),
            pltpu.SemaphoreType.DMA((NBUF,)),
            pltpu.SemaphoreType.DMA((NBUF,)),
        ],
    )(_sc_body)
    out = f(x2, fsplat, tab)
    return out.reshape(B, S, d)


# speedup vs baseline: 1.4639x; 1.0025x over previous
import functools

import jax
import jax.numpy as jnp
from jax import lax
from jax.experimental import pallas as pl
from jax.experimental.pallas import tpu as pltpu
from jax.experimental.pallas import tpu_sc as plsc

D = 1024
N_ROWS = 16384
NW = 32
ROWS_PER_W = N_ROWS // NW
C = 16
N_CHUNKS = ROWS_PER_W // C
LANES = 16
CBLKS = D // LANES
NBUF = 6
PD = 4


def _sc_body(x_hbm, fs_hbm, tab_hbm, out_hbm,
             xbuf, sbuf, tbuf, semx, semi, semo):
    wid = lax.axis_index("s") * 2 + lax.axis_index("c")
    base = wid * ROWS_PER_W

    pltpu.sync_copy(tab_hbm, tbuf)

    def in_descs(g, b):
        row0 = base + g * C
        return (
            pltpu.make_async_copy(x_hbm.at[pl.ds(row0, C)], xbuf.at[b],
                                  semx.at[b]),
            pltpu.make_async_copy(fs_hbm.at[pl.ds(row0, C)], sbuf.at[b],
                                  semi.at[b]),
        )

    def out_desc(g, b):
        row0 = base + g * C
        return pltpu.make_async_copy(xbuf.at[b], out_hbm.at[pl.ds(row0, C)],
                                     semo.at[b])

    def start_in(g, b):
        for d in in_descs(g, b):
            d.start()

    def wait_in(g, b):
        for d in in_descs(g, b):
            d.wait()

    def compute(b):
        @plsc.parallel_loop(0, C, 2)
        def row_body(r):
            fa = sbuf[b, r]
            fb = sbuf[b, r + 1]

            @plsc.parallel_loop(0, CBLKS, 1, unroll=8)
            def blk_body(c):
                off = c * LANES
                ta = plsc.load_gather(tbuf, [fa + off])
                tb = plsc.load_gather(tbuf, [fb + off])
                plsc.addupdate(xbuf.at[b, r, pl.ds(off, LANES)], ta)
                plsc.addupdate(xbuf.at[b, r + 1, pl.ds(off, LANES)], tb)

    for g in range(PD):
        start_in(g, g % NBUF)
    for g in range(PD):
        b = g % NBUF
        wait_in(g, b)
        gp = g + PD
        bp = gp % NBUF
        if gp >= NBUF:
            out_desc(gp - NBUF, bp).wait()
        start_in(gp, bp)
        compute(b)
        out_desc(g, b).start()

    def main_body(kk, _):
        for off in range(NBUF):
            g = kk * NBUF + PD + off
            b = (PD + off) % NBUF
            bp = (PD + off + PD) % NBUF
            wait_in(g, b)
            out_desc(g + PD - NBUF, bp).wait()
            start_in(g + PD, bp)
            compute(b)
            out_desc(g, b).start()
        return 0

    lax.fori_loop(0, (N_CHUNKS - 2 * PD) // NBUF, main_body, 0)

    for g in range(N_CHUNKS - PD, N_CHUNKS):
        b = g % NBUF
        wait_in(g, b)
        compute(b)
        out_desc(g, b).start()

    for g in range(N_CHUNKS - NBUF, N_CHUNKS):
        out_desc(g, g % NBUF).wait()


def kernel(x, type_idx, type_embedding):
    B, S, d = x.shape
    x2 = x.reshape(N_ROWS, D)
    idx = type_idx.reshape(N_ROWS).astype(jnp.int32)
    fsplat = idx[:, None] * D + jnp.arange(LANES, dtype=jnp.int32)[None, :]
    tab = type_embedding.reshape(3 * D)

    mesh = plsc.VectorSubcoreMesh(core_axis_name="c", subcore_axis_name="s")
    f = functools.partial(
        pl.kernel,
        out_type=jax.ShapeDtypeStruct((N_ROWS, D), jnp.float32),
        mesh=mesh,
        compiler_params=pltpu.CompilerParams(
            needs_layout_passes=False, use_tc_tiling_on_sc=True),
        scratch_types=[
            pltpu.VMEM((NBUF, C, D), jnp.float32),
            pltpu.VMEM((NBUF, C, LANES), jnp.int32),
            pltpu.VMEM((3 * D,), jnp.float32),
            pltpu.SemaphoreType.DMA((NBUF,)),
            pltpu.SemaphoreType.DMA((NBUF,)),
            pltpu.SemaphoreType.DMA((NBUF,)),
        ],
    )(_sc_body)
    out = f(x2, fsplat, tab)
    return out.reshape(B, S, d)
